# Initial kernel scaffold; baseline (speedup 1.0000x reference)
import jax
import jax.numpy as jnp
from jax.experimental import pallas as pl


def kernel(cell_feature, gene_feature, peak_feature, node_type, edge_index, edge_type, W_emb, b_emb, W_adapt, b_adapt, W_rel, ln_gamma, ln_beta):
    raise NotImplementedError("write your pallas kernel here")


# speedup vs baseline: 1.9535x; 1.9535x over previous
import functools

import jax
import jax.numpy as jnp
from jax import lax
from jax.experimental import pallas as pl
from jax.experimental.pallas import tpu as pltpu
from jax.experimental.pallas import tpu_sc as plsc

N_CELL, N_GENE, N_PEAK = 25000, 15000, 10000
N = N_CELL + N_GENE + N_PEAK
D_IN, D_EMB, H = 512, 256, 64
E = 800000
R = 6
L = 2

CH = 128
E_PAD = 32 * 196 * CH
T_ROWS = 50048
ROWS_PT = T_ROWS // 16
SB = 256
CHUNKS = tuple((k * SB, SB) for k in range(12)) + ((12 * SB, 56),)
DUMMY = N

BN = 1000


def _encode_body(x_ref, w1_ref, b1_ref, w2_ref, b2_ref, o_ref):
    h = jax.nn.relu(
        jnp.dot(x_ref[...], w1_ref[...], preferred_element_type=jnp.float32)
        + b1_ref[...])
    o_ref[...] = jnp.tanh(
        jnp.dot(h, w2_ref[...], preferred_element_type=jnp.float32)
        + b2_ref[...])


def _encode(feat, w1, b1, w2, b2):
    n = feat.shape[0]
    return pl.pallas_call(
        _encode_body,
        grid=(n // BN,),
        in_specs=[
            pl.BlockSpec((BN, D_IN), lambda i: (i, 0)),
            pl.BlockSpec((D_IN, D_EMB), lambda i: (0, 0)),
            pl.BlockSpec((1, D_EMB), lambda i: (0, 0)),
            pl.BlockSpec((D_EMB, H), lambda i: (0, 0)),
            pl.BlockSpec((1, H), lambda i: (0, 0)),
        ],
        out_specs=pl.BlockSpec((BN, H), lambda i: (i, 0)),
        out_shape=jax.ShapeDtypeStruct((n, H), jnp.float32),
    )(feat, w1, b1.reshape(1, D_EMB), w2, b2.reshape(1, H))


def _relmm_body(x_ref, w_ref, o_ref):
    x = x_ref[...]
    for r in range(R):
        o_ref[r] = jnp.dot(x, w_ref[r], preferred_element_type=jnp.float32)


def _relmm(x, w_l):
    return pl.pallas_call(
        _relmm_body,
        grid=(N // BN,),
        in_specs=[
            pl.BlockSpec((BN, H), lambda i: (i, 0)),
            pl.BlockSpec((R, H, H), lambda i: (0, 0, 0)),
        ],
        out_specs=pl.BlockSpec((R, BN, H), lambda i: (0, i, 0)),
        out_shape=jax.ShapeDtypeStruct((R, N, H), jnp.float32),
    )(x, w_l)


def _update_body(x_ref, a_ref, d_ref, g_ref, b_ref, o_ref):
    agg = jnp.concatenate([a_ref[0], a_ref[1]], axis=-1)
    deg = d_ref[0, :, :1] + d_ref[1, :, :1]
    scale = 1.0 / jnp.maximum(deg, 1.0)
    y = x_ref[...] + jax.nn.gelu(agg * scale)
    mu = jnp.mean(y, axis=-1, keepdims=True)
    var = jnp.mean((y - mu) ** 2, axis=-1, keepdims=True)
    o_ref[...] = (y - mu) * lax.rsqrt(var + 1e-5) * g_ref[...] + b_ref[...]


def _update(x, agg2, deg2, gamma, beta):
    return pl.pallas_call(
        _update_body,
        grid=(N // BN,),
        in_specs=[
            pl.BlockSpec((BN, H), lambda i: (i, 0)),
            pl.BlockSpec((2, BN, 32), lambda i: (0, i, 0)),
            pl.BlockSpec((2, BN, 16), lambda i: (0, i, 0)),
            pl.BlockSpec((1, H), lambda i: (0, 0)),
            pl.BlockSpec((1, H), lambda i: (0, 0)),
        ],
        out_specs=pl.BlockSpec((BN, H), lambda i: (i, 0)),
        out_shape=jax.ShapeDtypeStruct((N, H), jnp.float32),
    )(x, agg2, deg2, gamma.reshape(1, H), beta.reshape(1, H))


@functools.cache
def _sc_agg_call():
    mesh = plsc.VectorSubcoreMesh(
        core_axis_name="c", subcore_axis_name="s",
        num_cores=2, num_subcores=16)
    return pl.kernel(
        _sc_agg_body,
        out_type=jax.ShapeDtypeStruct((2, T_ROWS, 32), jnp.float32),
        mesh=mesh,
        scratch_types=[
            pltpu.VMEM((CH,), jnp.int32),
            pltpu.VMEM((CH,), jnp.int32),
            pltpu.VMEM((CH, 32), jnp.float32),
            pltpu.VMEM((SB, 32), jnp.float32),
            pltpu.VMEM_SHARED((T_ROWS, 32), jnp.float32),
            pltpu.SemaphoreType.DMA,
        ],
        compiler_params=pltpu.CompilerParams(use_tc_tiling_on_sc=False),
    )


def _sc_agg_body(xr2_hbm, eidx_a_hbm, eidx_b_hbm, dst_hbm, zeros_hbm,
                 out_hbm, idx_g, idx_s, rows, stage, table, sem):
    c = lax.axis_index("c")
    s = lax.axis_index("s")
    pltpu.sync_copy(zeros_hbm, stage)
    for off, nr in CHUNKS:
        pltpu.sync_copy(stage.at[pl.ds(0, nr)],
                        table.at[pl.ds(s * ROWS_PT + off, nr)])
    plsc.subcore_barrier()

    ept = E_PAD // 16
    nchunk = ept // CH

    def step(j, carry):
        base = s * ept + j * CH

        @pl.when(c == 0)
        def _():
            pltpu.sync_copy(eidx_a_hbm.at[pl.ds(base, CH)], idx_g)

        @pl.when(c == 1)
        def _():
            pltpu.sync_copy(eidx_b_hbm.at[pl.ds(base, CH)], idx_g)

        pltpu.sync_copy(dst_hbm.at[pl.ds(base, CH)], idx_s)
        pltpu.async_copy(xr2_hbm.at[idx_g], rows, sem).wait()
        pltpu.sync_copy(rows, table.at[idx_s], add=True)
        return carry

    lax.fori_loop(0, nchunk, step, 0)
    plsc.subcore_barrier()

    for off, nr in CHUNKS:
        r0 = s * ROWS_PT + off
        pltpu.sync_copy(table.at[pl.ds(r0, nr)], stage.at[pl.ds(0, nr)])
        pltpu.sync_copy(stage.at[pl.ds(0, nr)], out_hbm.at[c, pl.ds(r0, nr)])


@functools.cache
def _sc_deg_call():
    mesh = plsc.VectorSubcoreMesh(
        core_axis_name="c", subcore_axis_name="s",
        num_cores=2, num_subcores=16)
    return pl.kernel(
        _sc_deg_body,
        out_type=jax.ShapeDtypeStruct((2, T_ROWS, 16), jnp.float32),
        mesh=mesh,
        scratch_types=[
            pltpu.VMEM((CH,), jnp.int32),
            pltpu.VMEM((CH, 16), jnp.float32),
            pltpu.VMEM((SB, 16), jnp.float32),
            pltpu.VMEM_SHARED((T_ROWS, 16), jnp.float32),
        ],
        compiler_params=pltpu.CompilerParams(use_tc_tiling_on_sc=False),
    )


def _sc_deg_body(dst_hbm, ones_hbm, zeros_hbm, out_hbm,
                 idx_s, ones_v, stage, table):
    c = lax.axis_index("c")
    s = lax.axis_index("s")
    pltpu.sync_copy(zeros_hbm, stage)
    for off, nr in CHUNKS:
        pltpu.sync_copy(stage.at[pl.ds(0, nr)],
                        table.at[pl.ds(s * ROWS_PT + off, nr)])
    pltpu.sync_copy(ones_hbm, ones_v)
    plsc.subcore_barrier()

    epw = E_PAD // 32
    nchunk = epw // CH

    def step(j, carry):
        base = (c * 16 + s) * epw + j * CH
        pltpu.sync_copy(dst_hbm.at[pl.ds(base, CH)], idx_s)
        pltpu.sync_copy(ones_v, table.at[idx_s], add=True)
        return carry

    lax.fori_loop(0, nchunk, step, 0)
    plsc.subcore_barrier()

    for off, nr in CHUNKS:
        r0 = s * ROWS_PT + off
        pltpu.sync_copy(table.at[pl.ds(r0, nr)], stage.at[pl.ds(0, nr)])
        pltpu.sync_copy(stage.at[pl.ds(0, nr)], out_hbm.at[c, pl.ds(r0, nr)])


def kernel(cell_feature, gene_feature, peak_feature, node_type, edge_index,
           edge_type, W_emb, b_emb, W_adapt, b_adapt, W_rel, ln_gamma,
           ln_beta):
    del node_type
    src = edge_index[0]
    dst = edge_index[1]

    base2 = 2 * (edge_type * N + src)
    eidx_a = jnp.pad(base2, (0, E_PAD - E))
    eidx_b = jnp.pad(base2 + 1, (0, E_PAD - E))
    dstp = jnp.pad(dst, (0, E_PAD - E), constant_values=DUMMY)
    zeros32 = jnp.zeros((SB, 32), jnp.float32)
    zeros16 = jnp.zeros((SB, 16), jnp.float32)
    ones16 = jnp.ones((CH, 16), jnp.float32)

    x = jnp.concatenate([
        _encode(cell_feature, W_emb[0], b_emb[0], W_adapt[0], b_adapt[0]),
        _encode(gene_feature, W_emb[1], b_emb[1], W_adapt[1], b_adapt[1]),
        _encode(peak_feature, W_emb[2], b_emb[2], W_adapt[2], b_adapt[2]),
    ], axis=0)

    deg2 = _sc_deg_call()(dstp, ones16, zeros16)

    for l in range(L):
        xr = _relmm(x, W_rel[l])
        xr2 = xr.reshape(R * N * 2, 32)
        agg2 = _sc_agg_call()(xr2, eidx_a, eidx_b, dstp, zeros32)
        x = _update(x, agg2, deg2, ln_gamma[l], ln_beta[l])
    return x


# speedup vs baseline: 3.5400x; 1.8121x over previous
import functools

import jax
import jax.numpy as jnp
from jax import lax
from jax.experimental import pallas as pl
from jax.experimental.pallas import tpu as pltpu
from jax.experimental.pallas import tpu_sc as plsc

N_CELL, N_GENE, N_PEAK = 25000, 15000, 10000
N = N_CELL + N_GENE + N_PEAK
D_IN, D_EMB, H = 512, 256, 64
E = 800000
R = 6
L = 2

CH = 128
E_PAD = 32 * 196 * CH
T_ROWS = 50048
ROWS_PT = T_ROWS // 16
SB = 128
CHUNKS = tuple((k * SB, SB) for k in range(24)) + ((24 * SB, 56),)
DUMMY = N

BN = 1000


def _encode_body(x_ref, w1_ref, b1_ref, w2_ref, b2_ref, o_ref):
    h = jax.nn.relu(
        jnp.dot(x_ref[...], w1_ref[...], preferred_element_type=jnp.float32)
        + b1_ref[...])
    o_ref[...] = jnp.tanh(
        jnp.dot(h, w2_ref[...], preferred_element_type=jnp.float32)
        + b2_ref[...])


def _encode(feat, w1, b1, w2, b2):
    n = feat.shape[0]
    return pl.pallas_call(
        _encode_body,
        grid=(n // BN,),
        in_specs=[
            pl.BlockSpec((BN, D_IN), lambda i: (i, 0)),
            pl.BlockSpec((D_IN, D_EMB), lambda i: (0, 0)),
            pl.BlockSpec((1, D_EMB), lambda i: (0, 0)),
            pl.BlockSpec((D_EMB, H), lambda i: (0, 0)),
            pl.BlockSpec((1, H), lambda i: (0, 0)),
        ],
        out_specs=pl.BlockSpec((BN, H), lambda i: (i, 0)),
        out_shape=jax.ShapeDtypeStruct((n, H), jnp.float32),
    )(feat, w1, b1.reshape(1, D_EMB), w2, b2.reshape(1, H))


def _relmm_body(x_ref, w_ref, o_ref):
    x = x_ref[...]
    for r in range(R):
        o_ref[r] = jnp.dot(x, w_ref[r], preferred_element_type=jnp.float32)


def _relmm(x, w_l):
    return pl.pallas_call(
        _relmm_body,
        grid=(N // BN,),
        in_specs=[
            pl.BlockSpec((BN, H), lambda i: (i, 0)),
            pl.BlockSpec((R, H, H), lambda i: (0, 0, 0)),
        ],
        out_specs=pl.BlockSpec((R, BN, H), lambda i: (0, i, 0)),
        out_shape=jax.ShapeDtypeStruct((R, N, H), jnp.float32),
    )(x, w_l)


def _update_body(x_ref, a_ref, d_ref, g_ref, b_ref, o_ref):
    agg = jnp.concatenate([a_ref[0], a_ref[1]], axis=-1)
    deg = d_ref[0, :, :1] + d_ref[1, :, :1]
    scale = 1.0 / jnp.maximum(deg, 1.0)
    y = x_ref[...] + jax.nn.gelu(agg * scale)
    mu = jnp.mean(y, axis=-1, keepdims=True)
    var = jnp.mean((y - mu) ** 2, axis=-1, keepdims=True)
    o_ref[...] = (y - mu) * lax.rsqrt(var + 1e-5) * g_ref[...] + b_ref[...]


def _update(x, agg2, deg2, gamma, beta):
    return pl.pallas_call(
        _update_body,
        grid=(N // BN,),
        in_specs=[
            pl.BlockSpec((BN, H), lambda i: (i, 0)),
            pl.BlockSpec((2, BN, 32), lambda i: (0, i, 0)),
            pl.BlockSpec((2, BN, 16), lambda i: (0, i, 0)),
            pl.BlockSpec((1, H), lambda i: (0, 0)),
            pl.BlockSpec((1, H), lambda i: (0, 0)),
        ],
        out_specs=pl.BlockSpec((BN, H), lambda i: (i, 0)),
        out_shape=jax.ShapeDtypeStruct((N, H), jnp.float32),
    )(x, agg2, deg2, gamma.reshape(1, H), beta.reshape(1, H))


BLK = 8
EPT = E_PAD // 16
NBLK = EPT // (BLK * CH)


@functools.cache
def _sc_agg_call():
    mesh = plsc.VectorSubcoreMesh(
        core_axis_name="c", subcore_axis_name="s",
        num_cores=2, num_subcores=16)
    return pl.kernel(
        _sc_agg_body,
        out_type=jax.ShapeDtypeStruct((2, T_ROWS, 32), jnp.float32),
        mesh=mesh,
        scratch_types=[
            pltpu.VMEM((2, BLK, CH), jnp.int32),
            pltpu.VMEM((2, BLK, CH), jnp.int32),
            pltpu.VMEM((4, CH, 32), jnp.float32),
            pltpu.VMEM_SHARED((T_ROWS, 32), jnp.float32),
            [pltpu.SemaphoreType.DMA] * 4,
            [pltpu.SemaphoreType.DMA] * 2,
        ],
        compiler_params=pltpu.CompilerParams(use_tc_tiling_on_sc=False),
    )


def _sc_agg_body(xr2_hbm, eidx_a_hbm, eidx_b_hbm, dst_hbm, zeros_hbm,
                 out_hbm, gi, si, rows, table, gsems, isems):
    c = lax.axis_index("c")
    s = lax.axis_index("s")
    pltpu.sync_copy(zeros_hbm, rows.at[0])
    for off, nr in CHUNKS:
        pltpu.sync_copy(rows.at[0, pl.ds(0, nr)],
                        table.at[pl.ds(s * ROWS_PT + off, nr)])
    plsc.subcore_barrier()

    def load_idx(b, p):
        row0 = s * (EPT // CH) + b * BLK

        @pl.when(c == 0)
        def _():
            pltpu.async_copy(eidx_a_hbm.at[pl.ds(row0, BLK)],
                             gi.at[p], isems[p])

        @pl.when(c == 1)
        def _():
            pltpu.async_copy(eidx_b_hbm.at[pl.ds(row0, BLK)],
                             gi.at[p], isems[p])

        pltpu.async_copy(dst_hbm.at[pl.ds(row0, BLK)], si.at[p], isems[p])

    def wait_idx(p):
        pltpu.make_async_copy(dst_hbm.at[pl.ds(0, BLK)],
                              gi.at[p], isems[p]).wait()
        pltpu.make_async_copy(dst_hbm.at[pl.ds(0, BLK)],
                              si.at[p], isems[p]).wait()

    def do_block(p):
        for k in range(4):
            pltpu.async_copy(xr2_hbm.at[gi.at[p, k]], rows.at[k], gsems[k])
        for k in range(BLK):
            pltpu.make_async_copy(
                xr2_hbm.at[gi.at[p, k % 4]], rows.at[k % 4],
                gsems[k % 4]).wait()
            if k + 4 < BLK:
                pltpu.async_copy(xr2_hbm.at[gi.at[p, k + 4]],
                                 rows.at[(k + 4) % 4], gsems[(k + 4) % 4])
            pltpu.sync_copy(rows.at[k % 4], table.at[si.at[p, k]], add=True)

    load_idx(0, 0)

    def outer(j, carry):
        wait_idx(0)
        load_idx(2 * j + 1, 1)
        do_block(0)
        wait_idx(1)
        load_idx(2 * j + 2, 0)
        do_block(1)
        return carry

    lax.fori_loop(0, NBLK // 2, outer, 0)
    wait_idx(0)
    do_block(0)
    plsc.subcore_barrier()

    for off, nr in CHUNKS:
        r0 = s * ROWS_PT + off
        pltpu.sync_copy(table.at[pl.ds(r0, nr)], rows.at[0, pl.ds(0, nr)])
        pltpu.sync_copy(rows.at[0, pl.ds(0, nr)], out_hbm.at[c, pl.ds(r0, nr)])


@functools.cache
def _sc_deg_call():
    mesh = plsc.VectorSubcoreMesh(
        core_axis_name="c", subcore_axis_name="s",
        num_cores=2, num_subcores=16)
    return pl.kernel(
        _sc_deg_body,
        out_type=jax.ShapeDtypeStruct((2, T_ROWS, 16), jnp.float32),
        mesh=mesh,
        scratch_types=[
            pltpu.VMEM((CH,), jnp.int32),
            pltpu.VMEM((CH, 16), jnp.float32),
            pltpu.VMEM((SB, 16), jnp.float32),
            pltpu.VMEM_SHARED((T_ROWS, 16), jnp.float32),
        ],
        compiler_params=pltpu.CompilerParams(use_tc_tiling_on_sc=False),
    )


def _sc_deg_body(dst_hbm, ones_hbm, zeros_hbm, out_hbm,
                 idx_s, ones_v, stage, table):
    c = lax.axis_index("c")
    s = lax.axis_index("s")
    pltpu.sync_copy(zeros_hbm, stage)
    for off, nr in CHUNKS:
        pltpu.sync_copy(stage.at[pl.ds(0, nr)],
                        table.at[pl.ds(s * ROWS_PT + off, nr)])
    pltpu.sync_copy(ones_hbm, ones_v)
    plsc.subcore_barrier()

    epw = E_PAD // 32
    nchunk = epw // CH

    def step(j, carry):
        base = (c * 16 + s) * epw + j * CH
        pltpu.sync_copy(dst_hbm.at[pl.ds(base, CH)], idx_s)
        pltpu.sync_copy(ones_v, table.at[idx_s], add=True)
        return carry

    lax.fori_loop(0, nchunk, step, 0)
    plsc.subcore_barrier()

    for off, nr in CHUNKS:
        r0 = s * ROWS_PT + off
        pltpu.sync_copy(table.at[pl.ds(r0, nr)], stage.at[pl.ds(0, nr)])
        pltpu.sync_copy(stage.at[pl.ds(0, nr)], out_hbm.at[c, pl.ds(r0, nr)])


def kernel(cell_feature, gene_feature, peak_feature, node_type, edge_index,
           edge_type, W_emb, b_emb, W_adapt, b_adapt, W_rel, ln_gamma,
           ln_beta):
    del node_type
    src = edge_index[0]
    dst = edge_index[1]

    base2 = 2 * (edge_type * N + src)
    eidx_a = jnp.pad(base2, (0, E_PAD - E)).reshape(-1, CH)
    eidx_b = jnp.pad(base2 + 1, (0, E_PAD - E)).reshape(-1, CH)
    dstp = jnp.pad(dst, (0, E_PAD - E), constant_values=DUMMY)
    dst2 = dstp.reshape(-1, CH)
    zeros32 = jnp.zeros((SB, 32), jnp.float32)
    zeros16 = jnp.zeros((SB, 16), jnp.float32)
    ones16 = jnp.ones((CH, 16), jnp.float32)

    x = jnp.concatenate([
        _encode(cell_feature, W_emb[0], b_emb[0], W_adapt[0], b_adapt[0]),
        _encode(gene_feature, W_emb[1], b_emb[1], W_adapt[1], b_adapt[1]),
        _encode(peak_feature, W_emb[2], b_emb[2], W_adapt[2], b_adapt[2]),
    ], axis=0)

    deg2 = _sc_deg_call()(dstp, ones16, zeros16)

    for l in range(L):
        xr = _relmm(x, W_rel[l])
        xr2 = xr.reshape(R * N * 2, 32)
        agg2 = _sc_agg_call()(xr2, eidx_a, eidx_b, dst2, zeros32)
        x = _update(x, agg2, deg2, ln_gamma[l], ln_beta[l])
    return x
